# Initial kernel scaffold; baseline (speedup 1.0000x reference)
import functools

import jax
import jax.numpy as jnp
from jax import lax
from jax.experimental import pallas as pl
from jax.experimental.pallas import tpu as pltpu
import jax.experimental.pallas.tpu_sc as plsc

N = 10000
E = 320000
D = 128
NC = 2
NS = 16
NW = NC * NS
EB = 128
ITERS = -(-E // (NW * EB))
EP_W = ITERS * EB
E_PAD = NW * EP_W
N_PAD = 10016
RPT = N_PAD // NS

BM = 2000
GRID = N // BM


def _sc_mesh():
    return plsc.VectorSubcoreMesh(core_axis_name="c", subcore_axis_name="s")


@functools.partial(
    pl.kernel,
    out_type=jax.ShapeDtypeStruct((NC, N_PAD, 16), jnp.float32),
    mesh=_sc_mesh(),
    scratch_types=[
        pltpu.VMEM((EB,), jnp.int32),
        pltpu.VMEM((EB, 16), jnp.float32),
        pltpu.VMEM_SHARED((N_PAD, 16), jnp.float32),
        pltpu.SemaphoreType.DMA,
    ],
)
def _deg_kernel(row_hbm, ones_hbm, zeros_hbm, out_hbm, rowv, onesv, acc, sem):
    c = lax.axis_index("c")
    s = lax.axis_index("s")
    wid = s * NC + c
    pltpu.sync_copy(zeros_hbm, acc.at[pl.ds(s * RPT, RPT)])
    pltpu.sync_copy(ones_hbm, onesv)
    plsc.subcore_barrier()

    def body(i, carry):
        off = wid * EP_W + i * EB
        pltpu.sync_copy(row_hbm.at[pl.ds(off, EB)], rowv)
        pltpu.sync_copy(onesv, acc.at[rowv], add=True)
        return carry

    lax.fori_loop(0, ITERS, body, 0)
    plsc.subcore_barrier()
    pltpu.sync_copy(acc.at[pl.ds(s * RPT, RPT)],
                    out_hbm.at[c, pl.ds(s * RPT, RPT)])


@functools.partial(
    pl.kernel,
    out_type=jax.ShapeDtypeStruct((NC, N_PAD, D), jnp.float32),
    mesh=_sc_mesh(),
    scratch_types=[
        pltpu.VMEM((EB,), jnp.int32),
        pltpu.VMEM((EB,), jnp.int32),
        pltpu.VMEM((EB, D), jnp.float32),
        pltpu.VMEM_SHARED((N_PAD, D), jnp.float32),
        pltpu.SemaphoreType.DMA,
    ],
)
def _agg_kernel(hs_hbm, row_hbm, col_hbm, zeros_hbm, out_hbm,
                rowv, colv, gbuf, acc, sem):
    c = lax.axis_index("c")
    s = lax.axis_index("s")
    wid = s * NC + c
    pltpu.sync_copy(zeros_hbm, acc.at[pl.ds(s * RPT, RPT)])
    plsc.subcore_barrier()

    def body(i, carry):
        off = wid * EP_W + i * EB
        pltpu.sync_copy(row_hbm.at[pl.ds(off, EB)], rowv)
        pltpu.sync_copy(col_hbm.at[pl.ds(off, EB)], colv)
        pltpu.async_copy(hs_hbm.at[colv], gbuf, sem).wait()
        pltpu.sync_copy(gbuf, acc.at[rowv], add=True)
        return carry

    lax.fori_loop(0, ITERS, body, 0)
    plsc.subcore_barrier()
    pltpu.sync_copy(acc.at[pl.ds(s * RPT, RPT)],
                    out_hbm.at[c, pl.ds(s * RPT, RPT)])


def _prep_body(d0_ref, d1_ref, x_ref, dis_ref, h0s_ref):
    deg = d0_ref[:, 0:1] + d1_ref[:, 0:1]
    dis = jnp.where(deg > 0, lax.rsqrt(jnp.maximum(deg, 1e-12)), 0.0)
    dis_b = jnp.broadcast_to(dis, (BM, D))
    dis_ref[...] = dis_b
    h0s_ref[...] = x_ref[...] * dis_b


def _prep_call(d0, d1, x):
    return pl.pallas_call(
        _prep_body,
        grid=(GRID,),
        in_specs=[
            pl.BlockSpec((BM, 16), lambda i: (i, 0)),
            pl.BlockSpec((BM, 16), lambda i: (i, 0)),
            pl.BlockSpec((BM, D), lambda i: (i, 0)),
        ],
        out_specs=[
            pl.BlockSpec((BM, D), lambda i: (i, 0)),
            pl.BlockSpec((BM, D), lambda i: (i, 0)),
        ],
        out_shape=[
            jax.ShapeDtypeStruct((N, D), jnp.float32),
            jax.ShapeDtypeStruct((N, D), jnp.float32),
        ],
    )(d0, d1, x)


def _leaky(v):
    return jnp.maximum(v, 0.2 * v)


def _dense_body(p0_ref, p1_ref, h_ref, dis_ref, wg_ref, bg_ref, wi_ref,
                bi_ref, hn_ref, hns_ref):
    dis = dis_ref[...]
    ha = (p0_ref[...] + p1_ref[...]) * dis
    h = h_ref[...]
    a = _leaky(jnp.dot(ha, wg_ref[...],
                       preferred_element_type=jnp.float32) + bg_ref[...])
    b = _leaky(jnp.dot(h * ha, wi_ref[...],
                       preferred_element_type=jnp.float32) + bi_ref[...])
    u = a + b
    sq = jnp.sum(u * u, axis=-1, keepdims=True)
    hn = u * lax.rsqrt(jnp.maximum(sq, 1e-12))
    hn_ref[...] = hn
    hns_ref[...] = hn * dis


def _dense_call(p0, p1, h, dis, wg, bg, wi, bi):
    full = lambda i: (0, 0)
    blk = lambda i: (i, 0)
    return pl.pallas_call(
        _dense_body,
        grid=(GRID,),
        in_specs=[
            pl.BlockSpec((BM, D), blk),
            pl.BlockSpec((BM, D), blk),
            pl.BlockSpec((BM, D), blk),
            pl.BlockSpec((BM, D), blk),
            pl.BlockSpec((D, D), full),
            pl.BlockSpec((1, D), full),
            pl.BlockSpec((D, D), full),
            pl.BlockSpec((1, D), full),
        ],
        out_specs=[
            pl.BlockSpec((BM, D), blk),
            pl.BlockSpec((BM, D), blk),
        ],
        out_shape=[
            jax.ShapeDtypeStruct((N, D), jnp.float32),
            jax.ShapeDtypeStruct((N, D), jnp.float32),
        ],
    )(p0, p1, h, dis, wg, bg, wi, bi)


@jax.jit
def _run(x, edge_index, Wg0, bg0, Wi0, bi0, Wg1, bg1, Wi1, bi1,
         Wg2, bg2, Wi2, bi2):
    row = edge_index[0]
    col = edge_index[1]
    pad = E_PAD - E
    row_p = jnp.concatenate([row, jnp.full((pad,), N, jnp.int32)])
    col_p = jnp.concatenate([col, jnp.zeros((pad,), jnp.int32)])

    zeros16 = jnp.zeros((RPT, 16), jnp.float32)
    zerosD = jnp.zeros((RPT, D), jnp.float32)
    ones16 = jnp.ones((EB, 16), jnp.float32)

    deg_parts = _deg_kernel(row_p, ones16, zeros16)
    dis, hs = _prep_call(deg_parts[0, :N], deg_parts[1, :N], x)

    params = [(Wg0, bg0, Wi0, bi0), (Wg1, bg1, Wi1, bi1), (Wg2, bg2, Wi2, bi2)]
    h = x
    outs = [x]
    for (Wg, bg, Wi, bi) in params:
        parts = _agg_kernel(hs, row_p, col_p, zerosD)
        h, hs = _dense_call(parts[0, :N], parts[1, :N], h, dis,
                            Wg, bg.reshape(1, D), Wi, bi.reshape(1, D))
        outs.append(h)
    return jnp.concatenate(outs, axis=-1)


def kernel(x, edge_index, Wg0, bg0, Wi0, bi0, Wg1, bg1, Wi1, bi1,
           Wg2, bg2, Wi2, bi2):
    return _run(x, edge_index, Wg0, bg0, Wi0, bi0, Wg1, bg1, Wi1, bi1,
                Wg2, bg2, Wi2, bi2)


# speedup vs baseline: 6.3973x; 6.3973x over previous
import functools

import jax
import jax.numpy as jnp
from jax import lax
from jax.experimental import pallas as pl
from jax.experimental.pallas import tpu as pltpu
import jax.experimental.pallas.tpu_sc as plsc

N = 10000
E = 320000
D = 128
NC = 2
NS = 16
NW = NC * NS
EB = 128
ITERS = -(-E // (NW * EB))
EP_W = ITERS * EB
E_PAD = NW * EP_W
N_PAD = 10112
RPT = N_PAD // NS

BM = 2000
GRID = N // BM


def _sc_mesh():
    return plsc.VectorSubcoreMesh(core_axis_name="c", subcore_axis_name="s",
                                  num_cores=NC, num_subcores=NS)


@functools.lru_cache(maxsize=None)
def _make_deg_kernel():
    @functools.partial(
        pl.kernel,
        out_type=jax.ShapeDtypeStruct((NC, N_PAD, D), jnp.float32),
        mesh=_sc_mesh(),
        scratch_types=[
            pltpu.VMEM((EB,), jnp.int32),
            pltpu.VMEM((EB, D), jnp.float32),
            pltpu.VMEM_SHARED((N_PAD, D), jnp.float32),
            pltpu.SemaphoreType.DMA,
        ],
    )
    def _deg_kernel(row_hbm, ones_hbm, zeros_hbm, out_hbm,
                    rowv, onesv, acc, sem):
        c = lax.axis_index("c")
        s = lax.axis_index("s")
        wid = s * NC + c
        pltpu.sync_copy(zeros_hbm, acc.at[pl.ds(s * RPT, RPT)])
        pltpu.sync_copy(ones_hbm, onesv)
        plsc.subcore_barrier()

        def body(i, carry):
            off = wid * EP_W + i * EB
            pltpu.sync_copy(row_hbm.at[pl.ds(off, EB)], rowv)
            pltpu.sync_copy(onesv, acc.at[rowv], add=True)
            return carry

        lax.fori_loop(0, ITERS, body, 0)
        plsc.subcore_barrier()
        pltpu.sync_copy(acc.at[pl.ds(s * RPT, RPT)],
                        out_hbm.at[c, pl.ds(s * RPT, RPT)])

    return _deg_kernel


@functools.lru_cache(maxsize=None)
def _make_agg_kernel():
    @functools.partial(
        pl.kernel,
        out_type=jax.ShapeDtypeStruct((NC, N_PAD, D), jnp.float32),
        mesh=_sc_mesh(),
        scratch_types=[
            pltpu.VMEM((EB,), jnp.int32),
            pltpu.VMEM((EB,), jnp.int32),
            pltpu.VMEM((EB, D), jnp.float32),
            pltpu.VMEM_SHARED((N_PAD, D), jnp.float32),
            pltpu.SemaphoreType.DMA,
        ],
    )
    def _agg_kernel(hs_hbm, row_hbm, col_hbm, zeros_hbm, out_hbm,
                    rowv, colv, gbuf, acc, sem):
        c = lax.axis_index("c")
        s = lax.axis_index("s")
        wid = s * NC + c
        pltpu.sync_copy(zeros_hbm, acc.at[pl.ds(s * RPT, RPT)])
        plsc.subcore_barrier()

        def body(i, carry):
            off = wid * EP_W + i * EB
            pltpu.sync_copy(row_hbm.at[pl.ds(off, EB)], rowv)
            pltpu.sync_copy(col_hbm.at[pl.ds(off, EB)], colv)
            pltpu.async_copy(hs_hbm.at[colv], gbuf, sem).wait()
            pltpu.sync_copy(gbuf, acc.at[rowv], add=True)
            return carry

        lax.fori_loop(0, ITERS, body, 0)
        plsc.subcore_barrier()
        pltpu.sync_copy(acc.at[pl.ds(s * RPT, RPT)],
                        out_hbm.at[c, pl.ds(s * RPT, RPT)])

    return _agg_kernel


def _prep_body(d0_ref, d1_ref, x_ref, dis_ref, h0s_ref):
    deg = d0_ref[:, 0:1] + d1_ref[:, 0:1]
    dis = jnp.where(deg > 0, lax.rsqrt(jnp.maximum(deg, 1e-12)), 0.0)
    dis_b = jnp.broadcast_to(dis, (BM, D))
    dis_ref[...] = dis_b
    h0s_ref[...] = x_ref[...] * dis_b


def _prep_call(d0, d1, x):
    return pl.pallas_call(
        _prep_body,
        grid=(GRID,),
        in_specs=[
            pl.BlockSpec((BM, D), lambda i: (i, 0)),
            pl.BlockSpec((BM, D), lambda i: (i, 0)),
            pl.BlockSpec((BM, D), lambda i: (i, 0)),
        ],
        out_specs=[
            pl.BlockSpec((BM, D), lambda i: (i, 0)),
            pl.BlockSpec((BM, D), lambda i: (i, 0)),
        ],
        out_shape=[
            jax.ShapeDtypeStruct((N, D), jnp.float32),
            jax.ShapeDtypeStruct((N, D), jnp.float32),
        ],
    )(d0, d1, x)


def _leaky(v):
    return jnp.maximum(v, 0.2 * v)


def _dense_body(p0_ref, p1_ref, h_ref, dis_ref, wg_ref, bg_ref, wi_ref,
                bi_ref, hn_ref, hns_ref):
    dis = dis_ref[...]
    ha = (p0_ref[...] + p1_ref[...]) * dis
    h = h_ref[...]
    a = _leaky(jnp.dot(ha, wg_ref[...],
                       preferred_element_type=jnp.float32) + bg_ref[...])
    b = _leaky(jnp.dot(h * ha, wi_ref[...],
                       preferred_element_type=jnp.float32) + bi_ref[...])
    u = a + b
    sq = jnp.sum(u * u, axis=-1, keepdims=True)
    hn = u * lax.rsqrt(jnp.maximum(sq, 1e-12))
    hn_ref[...] = hn
    hns_ref[...] = hn * dis


def _dense_call(p0, p1, h, dis, wg, bg, wi, bi):
    full = lambda i: (0, 0)
    blk = lambda i: (i, 0)
    return pl.pallas_call(
        _dense_body,
        grid=(GRID,),
        in_specs=[
            pl.BlockSpec((BM, D), blk),
            pl.BlockSpec((BM, D), blk),
            pl.BlockSpec((BM, D), blk),
            pl.BlockSpec((BM, D), blk),
            pl.BlockSpec((D, D), full),
            pl.BlockSpec((1, D), full),
            pl.BlockSpec((D, D), full),
            pl.BlockSpec((1, D), full),
        ],
        out_specs=[
            pl.BlockSpec((BM, D), blk),
            pl.BlockSpec((BM, D), blk),
        ],
        out_shape=[
            jax.ShapeDtypeStruct((N, D), jnp.float32),
            jax.ShapeDtypeStruct((N, D), jnp.float32),
        ],
    )(p0, p1, h, dis, wg, bg, wi, bi)


@jax.jit
def _run(x, edge_index, Wg0, bg0, Wi0, bi0, Wg1, bg1, Wi1, bi1,
         Wg2, bg2, Wi2, bi2):
    row = edge_index[0]
    col = edge_index[1]
    pad = E_PAD - E
    row_p = jnp.concatenate([row, jnp.full((pad,), N, jnp.int32)])
    col_p = jnp.concatenate([col, jnp.zeros((pad,), jnp.int32)])

    zerosD = jnp.zeros((RPT, D), jnp.float32)
    onesD = jnp.ones((EB, D), jnp.float32)

    deg_parts = _make_deg_kernel()(row_p, onesD, zerosD)
    dis, hs = _prep_call(deg_parts[0, :N], deg_parts[1, :N], x)

    params = [(Wg0, bg0, Wi0, bi0), (Wg1, bg1, Wi1, bi1), (Wg2, bg2, Wi2, bi2)]
    h = x
    outs = [x]
    for (Wg, bg, Wi, bi) in params:
        parts = _make_agg_kernel()(hs, row_p, col_p, zerosD)
        h, hs = _dense_call(parts[0, :N], parts[1, :N], h, dis,
                            Wg, bg.reshape(1, D), Wi, bi.reshape(1, D))
        outs.append(h)
    return jnp.concatenate(outs, axis=-1)


def kernel(x, edge_index, Wg0, bg0, Wi0, bi0, Wg1, bg1, Wi1, bi1,
           Wg2, bg2, Wi2, bi2):
    return _run(x, edge_index, Wg0, bg0, Wi0, bi0, Wg1, bg1, Wi1, bi1,
                Wg2, bg2, Wi2, bi2)


# speedup vs baseline: 6.5205x; 1.0193x over previous
import functools

import jax
import jax.numpy as jnp
from jax import lax
from jax.experimental import pallas as pl
from jax.experimental.pallas import tpu as pltpu
import jax.experimental.pallas.tpu_sc as plsc

N = 10000
E = 320000
D = 128
NC = 2
NS = 16
NW = NC * NS
EB = 128
NBUF = 2
IB = 4
ITERS = 80
TGRP = ITERS // (2 * NBUF)
EP_W = ITERS * EB
E_PAD = NW * EP_W
N_PAD = 10112
RPT = N_PAD // NS

BM = 2000
GRID = N // BM


def _sc_mesh():
    return plsc.VectorSubcoreMesh(core_axis_name="c", subcore_axis_name="s",
                                  num_cores=NC, num_subcores=NS)


@functools.lru_cache(maxsize=None)
def _make_deg_kernel():
    @functools.partial(
        pl.kernel,
        out_type=jax.ShapeDtypeStruct((NC, N_PAD, D), jnp.float32),
        mesh=_sc_mesh(),
        scratch_types=[
            pltpu.VMEM((ITERS, EB), jnp.int32),
            pltpu.VMEM((EB, D), jnp.float32),
            pltpu.VMEM_SHARED((N_PAD, D), jnp.float32),
            pltpu.SemaphoreType.DMA((IB,)),
        ],
    )
    def _deg_kernel(row_hbm, ones_hbm, zeros_hbm, out_hbm,
                    rowm, onesv, acc, ssem):
        c = lax.axis_index("c")
        s = lax.axis_index("s")
        wid = s * NC + c
        pltpu.sync_copy(row_hbm.at[wid], rowm)
        pltpu.sync_copy(zeros_hbm, acc.at[pl.ds(s * RPT, RPT)])
        pltpu.sync_copy(ones_hbm, onesv)
        plsc.subcore_barrier()

        def body(sup, carry):
            for b in range(IB):
                k = sup * IB + b
                pltpu.async_copy(onesv, acc.at[rowm.at[k]], ssem.at[b],
                                 add=True)
            for b in range(IB):
                k = sup * IB + b
                pltpu.make_async_copy(onesv, acc.at[rowm.at[k]],
                                      ssem.at[b]).wait()
            return carry

        lax.fori_loop(0, ITERS // IB, body, 0)
        plsc.subcore_barrier()
        pltpu.sync_copy(acc.at[pl.ds(s * RPT, RPT)],
                        out_hbm.at[c, pl.ds(s * RPT, RPT)])

    return _deg_kernel


@functools.lru_cache(maxsize=None)
def _make_agg_kernel():
    @functools.partial(
        pl.kernel,
        out_type=jax.ShapeDtypeStruct((NC, N_PAD, D), jnp.float32),
        mesh=_sc_mesh(),
        scratch_types=(
            [pltpu.VMEM((EB,), jnp.int32) for _ in range(2 * IB)]
            + [pltpu.VMEM((EB, D), jnp.float32) for _ in range(NBUF)]
            + [
                pltpu.VMEM_SHARED((N_PAD, D), jnp.float32),
                pltpu.SemaphoreType.DMA((IB,)),
                pltpu.SemaphoreType.DMA((NBUF,)),
                pltpu.SemaphoreType.DMA((NBUF,)),
            ]
        ),
    )
    def _agg_kernel(hs_hbm, row_hbm, col_hbm, zeros_hbm, out_hbm, *rest):
        rowb = rest[:IB]
        colb = rest[IB:2 * IB]
        gbufs = rest[2 * IB:2 * IB + NBUF]
        acc, isem, gsem, ssem = rest[2 * IB + NBUF:2 * IB + NBUF + 4]
        c = lax.axis_index("c")
        s = lax.axis_index("s")
        wid = s * NC + c

        def idx_start(k, j):
            pltpu.async_copy(row_hbm.at[wid, k], rowb[j], isem.at[j])
            pltpu.async_copy(col_hbm.at[wid, k], colb[j], isem.at[j])

        def idx_wait(k, j):
            pltpu.make_async_copy(row_hbm.at[wid, k], rowb[j],
                                  isem.at[j]).wait()
            pltpu.make_async_copy(col_hbm.at[wid, k], colb[j],
                                  isem.at[j]).wait()

        def gather_start(b, j):
            pltpu.async_copy(hs_hbm.at[colb[j]], gbufs[b], gsem.at[b])

        def gather_wait(b, j):
            pltpu.make_async_copy(hs_hbm.at[colb[j]], gbufs[b],
                                  gsem.at[b]).wait()

        def scat_start(b, j):
            pltpu.async_copy(gbufs[b], acc.at[rowb[j]], ssem.at[b], add=True)

        def scat_wait(b, j):
            pltpu.make_async_copy(gbufs[b], acc.at[rowb[j]],
                                  ssem.at[b]).wait()

        pltpu.sync_copy(zeros_hbm, acc.at[pl.ds(s * RPT, RPT)])
        plsc.subcore_barrier()

        for j in range(IB):
            idx_start(j, j)
        for b in range(NBUF):
            idx_wait(b, b)
            gather_start(b, b)

        def body(t, carry):
            k0 = 4 * t

            def half(base, j0, j1, jn0, jn1, refill_ok, prefetch_ok):
                gather_wait(0, j0)
                scat_start(0, j0)
                gather_wait(1, j1)
                scat_start(1, j1)

                def refill():
                    scat_wait(0, j0)
                    idx_wait(base + 2, jn0)
                    gather_start(0, jn0)
                    scat_wait(1, j1)
                    idx_wait(base + 3, jn1)
                    gather_start(1, jn1)

                if refill_ok is None:
                    refill()
                else:
                    pl.when(refill_ok)(refill)

                @pl.when(prefetch_ok)
                def _():
                    idx_start(base + 4, j0)
                    idx_start(base + 5, j1)

            not_last = t < TGRP - 1
            half(k0, 0, 1, 2, 3, None, not_last)
            half(k0 + 2, 2, 3, 0, 1, not_last, not_last)
            return carry

        lax.fori_loop(0, TGRP, body, 0)
        scat_wait(0, 2)
        scat_wait(1, 3)
        plsc.subcore_barrier()
        pltpu.sync_copy(acc.at[pl.ds(s * RPT, RPT)],
                        out_hbm.at[c, pl.ds(s * RPT, RPT)])

    return _agg_kernel


def _prep_body(d0_ref, d1_ref, x_ref, dis_ref, h0s_ref):
    deg = d0_ref[:, 0:1] + d1_ref[:, 0:1]
    dis = jnp.where(deg > 0, lax.rsqrt(jnp.maximum(deg, 1e-12)), 0.0)
    dis_b = jnp.broadcast_to(dis, (BM, D))
    dis_ref[...] = dis_b
    h0s_ref[...] = x_ref[...] * dis_b


def _prep_call(d0, d1, x):
    return pl.pallas_call(
        _prep_body,
        grid=(GRID,),
        in_specs=[
            pl.BlockSpec((BM, D), lambda i: (i, 0)),
            pl.BlockSpec((BM, D), lambda i: (i, 0)),
            pl.BlockSpec((BM, D), lambda i: (i, 0)),
        ],
        out_specs=[
            pl.BlockSpec((BM, D), lambda i: (i, 0)),
            pl.BlockSpec((BM, D), lambda i: (i, 0)),
        ],
        out_shape=[
            jax.ShapeDtypeStruct((N, D), jnp.float32),
            jax.ShapeDtypeStruct((N, D), jnp.float32),
        ],
    )(d0, d1, x)


def _leaky(v):
    return jnp.maximum(v, 0.2 * v)


def _dense_body(p0_ref, p1_ref, h_ref, dis_ref, wg_ref, bg_ref, wi_ref,
                bi_ref, hn_ref, hns_ref):
    dis = dis_ref[...]
    ha = (p0_ref[...] + p1_ref[...]) * dis
    h = h_ref[...]
    a = _leaky(jnp.dot(ha, wg_ref[...],
                       preferred_element_type=jnp.float32) + bg_ref[...])
    b = _leaky(jnp.dot(h * ha, wi_ref[...],
                       preferred_element_type=jnp.float32) + bi_ref[...])
    u = a + b
    sq = jnp.sum(u * u, axis=-1, keepdims=True)
    hn = u * lax.rsqrt(jnp.maximum(sq, 1e-12))
    hn_ref[...] = hn
    hns_ref[...] = hn * dis


def _dense_call(p0, p1, h, dis, wg, bg, wi, bi):
    full = lambda i: (0, 0)
    blk = lambda i: (i, 0)
    return pl.pallas_call(
        _dense_body,
        grid=(GRID,),
        in_specs=[
            pl.BlockSpec((BM, D), blk),
            pl.BlockSpec((BM, D), blk),
            pl.BlockSpec((BM, D), blk),
            pl.BlockSpec((BM, D), blk),
            pl.BlockSpec((D, D), full),
            pl.BlockSpec((1, D), full),
            pl.BlockSpec((D, D), full),
            pl.BlockSpec((1, D), full),
        ],
        out_specs=[
            pl.BlockSpec((BM, D), blk),
            pl.BlockSpec((BM, D), blk),
        ],
        out_shape=[
            jax.ShapeDtypeStruct((N, D), jnp.float32),
            jax.ShapeDtypeStruct((N, D), jnp.float32),
        ],
    )(p0, p1, h, dis, wg, bg, wi, bi)


@jax.jit
def _run(x, edge_index, Wg0, bg0, Wi0, bi0, Wg1, bg1, Wi1, bi1,
         Wg2, bg2, Wi2, bi2):
    row = edge_index[0]
    col = edge_index[1]
    pad = E_PAD - E
    row_p = jnp.concatenate([row, jnp.full((pad,), N, jnp.int32)])
    col_p = jnp.concatenate([col, jnp.zeros((pad,), jnp.int32)])
    row3 = row_p.reshape(NW, ITERS, EB)
    col3 = col_p.reshape(NW, ITERS, EB)

    zerosD = jnp.zeros((RPT, D), jnp.float32)
    onesD = jnp.ones((EB, D), jnp.float32)

    deg_parts = _make_deg_kernel()(row3, onesD, zerosD)
    dis, hs = _prep_call(deg_parts[0, :N], deg_parts[1, :N], x)

    params = [(Wg0, bg0, Wi0, bi0), (Wg1, bg1, Wi1, bi1), (Wg2, bg2, Wi2, bi2)]
    h = x
    outs = [x]
    for (Wg, bg, Wi, bi) in params:
        parts = _make_agg_kernel()(hs, row3, col3, zerosD)
        h, hs = _dense_call(parts[0, :N], parts[1, :N], h, dis,
                            Wg, bg.reshape(1, D), Wi, bi.reshape(1, D))
        outs.append(h)
    return jnp.concatenate(outs, axis=-1)


def kernel(x, edge_index, Wg0, bg0, Wi0, bi0, Wg1, bg1, Wi1, bi1,
           Wg2, bg2, Wi2, bi2):
    return _run(x, edge_index, Wg0, bg0, Wi0, bi0, Wg1, bg1, Wi1, bi1,
                Wg2, bg2, Wi2, bi2)
